# baseline (speedup 1.0000x reference)
import functools

import jax
import jax.numpy as jnp
import numpy as np
from jax.experimental import pallas as pl

N = 100000
E = 1600000
NF = 1
NU = 16
NB = 3
NL = 3
NSTEPS = 100
NTX = 10000


def _time_embed(t, dim):
    half = dim // 2
    freqs = jnp.exp(-np.log(10000.0) * jnp.arange(half) / half)
    args = t[:, None].astype(jnp.float32) * freqs[None, :]
    return jnp.concatenate([jnp.sin(args), jnp.cos(args)], axis=-1)


def _seg_softmax(scores, seg, num_segments):
    m = jax.ops.segment_max(scores, seg, num_segments=num_segments)
    m = jnp.where(jnp.isfinite(m), m, 0.0)
    ex = jnp.exp(scores - m[seg])
    denom = jax.ops.segment_sum(ex, seg, num_segments=num_segments)
    return ex / (denom[seg] + 1e-16)


def _copy_kernel(x_ref, o_ref):
    o_ref[...] = x_ref[...]


def kernel(x, t, edge_index, edge_weight, batch, transmitters_index,
           W_in, b_in, Wq, Wk, Wv, We, Wt, Wy, W1, W2, W_out, b_out):
    y = x
    val = x @ W_in + b_in
    temb = _time_embed(t, NU)
    ew2 = edge_weight[:, None]
    src, dst = edge_index[0], edge_index[1]
    scale = 1.0 / jnp.sqrt(float(NU))
    for b in range(NB):
        for l in range(NL):
            h = val + temb @ Wt[b, l] + y @ Wy[b, l]
            q = h @ Wq[b, l]
            k = h @ Wk[b, l]
            v = h @ Wv[b, l]
            ew = ew2 @ We[b, l]
            scores = jnp.sum(q[dst] * (k[src] + ew), axis=-1) * scale
            alpha = _seg_softmax(scores, dst, N)
            agg = jax.ops.segment_sum(alpha[:, None] * (v[src] + ew), dst, num_segments=N)
            h2 = val + agg
            ff = jax.nn.relu(h2 @ W1[b, l]) @ W2[b, l]
            val = h2 + ff
    val = val @ W_out + b_out
    sums = jax.ops.segment_sum(val, transmitters_index, num_segments=NTX)
    counts = jax.ops.segment_sum(jnp.ones((val.shape[0],), dtype=jnp.float32),
                                 transmitters_index, num_segments=NTX)
    Tx_embeddings = sums / jnp.maximum(counts, 1.0)[:, None]
    Tx_embeddings = pl.pallas_call(
        _copy_kernel,
        out_shape=jax.ShapeDtypeStruct((NTX, NF), jnp.float32),
    )(Tx_embeddings)
    return Tx_embeddings


# speedup vs baseline: 15.6480x; 15.6480x over previous
import functools

import jax
import jax.numpy as jnp
import numpy as np
from jax import lax
from jax.experimental import pallas as pl
from jax.experimental.pallas import tpu as pltpu
from jax.experimental.pallas import tpu_sc as plsc

N = 100000
E = 1600000
NF = 1
NU = 16
NB = 3
NL = 3
NSTEPS = 100
NTX = 10000

NW = 32
R = 3136
NP2 = NW * R
ACC = R + 16
DUMMY = R
ECAP = 65536
C = 512
CH0 = 2000
STG = 3072
NEG = -3.0e38

_SCALE = 1.0 / np.sqrt(float(NU))

_MESH = plsc.VectorSubcoreMesh(core_axis_name="c", subcore_axis_name="s")
_SC_PARAMS = pltpu.CompilerParams(needs_layout_passes=False, use_tc_tiling_on_sc=False)


def _wid():
    return lax.axis_index("s") * 2 + lax.axis_index("c")


def _al(x):
    return pl.multiple_of(x, 8)


@functools.partial(
    pl.kernel,
    mesh=_MESH,
    compiler_params=_SC_PARAMS,
    out_type=[
        jax.ShapeDtypeStruct((NW * ECAP,), jnp.int32),
        jax.ShapeDtypeStruct((NW * ECAP,), jnp.int32),
        jax.ShapeDtypeStruct((NW * ECAP,), jnp.float32),
        jax.ShapeDtypeStruct((NW * 16,), jnp.int32),
    ],
    scratch_types=[
        pltpu.VMEM((CH0,), jnp.int32),
        pltpu.VMEM((CH0,), jnp.int32),
        pltpu.VMEM((CH0,), jnp.float32),
        pltpu.VMEM((STG,), jnp.int32),
        pltpu.VMEM((STG,), jnp.int32),
        pltpu.VMEM((STG,), jnp.float32),
    ],
)
def _bucket_sc(src_hbm, dst_hbm, w_hbm, bsrc, bdst, bw, cnt_hbm,
               src_v, dst_v, w_v, ssrc, sdst, sw):
    wid = _wid()
    ebase = wid * ECAP
    lo = wid * R
    hi = jnp.where(wid < NW - 1, lo + R, N)
    iota = lax.iota(jnp.int32, 16)
    zi = jnp.zeros((16,), jnp.int32)
    zf = jnp.zeros((16,), jnp.float32)
    dumv = jnp.full((16,), DUMMY, jnp.int32)

    def flush(nfl, off):
        def fl(i, _):
            s = pl.ds(i * C, C)
            pltpu.sync_copy(ssrc.at[s], bsrc.at[pl.ds(_al(ebase + off + i * C), C)])
            pltpu.sync_copy(sdst.at[s], bdst.at[pl.ds(_al(ebase + off + i * C), C)])
            pltpu.sync_copy(sw.at[s], bw.at[pl.ds(_al(ebase + off + i * C), C)])
            return 0
        lax.fori_loop(0, nfl, fl, 0)

    def chunk(ch, carry):
        fill, off = carry
        s = pl.ds(ch * CH0, CH0)
        pltpu.sync_copy(src_hbm.at[s], src_v)
        pltpu.sync_copy(dst_hbm.at[s], dst_v)
        pltpu.sync_copy(w_hbm.at[s], w_v)

        def grp(g, fill):
            sl = pl.ds(g * 16, 16)
            d16 = dst_v[sl]
            own = (d16 >= lo) & (d16 < hi)
            onesi = jnp.where(own, 1, 0)
            pos = jnp.full((16,), fill, jnp.int32) + plsc.cumsum(onesi) - onesi
            plsc.store_scatter(ssrc, [pos], src_v[sl], mask=own)
            plsc.store_scatter(sdst, [pos], d16 - lo, mask=own)
            plsc.store_scatter(sw, [pos], w_v[sl], mask=own)
            return fill + jnp.sum(onesi)

        fill = lax.fori_loop(0, CH0 // 16, grp, fill)
        nfl = fill // C
        flush(nfl, off)
        off = off + nfl * C
        rem = fill - nfl * C

        def comp(i, _):
            lane = i * 16 + iota
            srcpos = nfl * C + lane
            msk = lane < rem
            vs = plsc.load_gather(ssrc, [srcpos], mask=msk)
            plsc.store_scatter(ssrc, [lane], vs, mask=msk)
            vd = plsc.load_gather(sdst, [srcpos], mask=msk)
            plsc.store_scatter(sdst, [lane], vd, mask=msk)
            vw = plsc.load_gather(sw, [srcpos], mask=msk)
            plsc.store_scatter(sw, [lane], vw, mask=msk)
            return 0

        lax.fori_loop(0, C // 16, comp, 0)
        return rem, off

    fill, off = lax.fori_loop(0, E // CH0, chunk, (jnp.int32(0), jnp.int32(0)))

    pad = (C - fill % C) % C

    def padstep(i, _):
        lane = i * 16 + iota
        msk = lane < pad
        fpos = jnp.full((16,), fill, jnp.int32) + lane
        plsc.store_scatter(ssrc, [fpos], zi, mask=msk)
        plsc.store_scatter(sdst, [fpos], dumv, mask=msk)
        plsc.store_scatter(sw, [fpos], zf, mask=msk)
        return 0

    lax.fori_loop(0, C // 16, padstep, 0)
    fill = fill + pad
    nfl = fill // C
    flush(nfl, off)
    off = off + nfl * C

    ssrc[pl.ds(0, 16)] = jnp.full((16,), off, jnp.int32)
    pltpu.sync_copy(ssrc.at[pl.ds(0, 16)], cnt_hbm.at[pl.ds(_al(wid * 16), 16)])


@functools.partial(
    pl.kernel,
    mesh=_MESH,
    compiler_params=_SC_PARAMS,
    out_type=[
        jax.ShapeDtypeStruct((NP2 * 16,), jnp.float32),
        jax.ShapeDtypeStruct((NP2,), jnp.float32),
        jax.ShapeDtypeStruct((NP2,), jnp.float32),
        jax.ShapeDtypeStruct((NW * ECAP,), jnp.float32),
    ],
    scratch_types=[
        pltpu.VMEM((ACC * 16,), jnp.float32),
        pltpu.VMEM((16 * ACC,), jnp.float32),
        pltpu.VMEM((ACC,), jnp.float32),
        pltpu.VMEM((ACC,), jnp.float32),
        pltpu.VMEM((C,), jnp.int32),
        pltpu.VMEM((C,), jnp.int32),
        pltpu.VMEM((C,), jnp.float32),
        pltpu.VMEM((C,), jnp.float32),
        pltpu.VMEM((C, 16), jnp.float32),
        pltpu.VMEM((16,), jnp.int32),
        pltpu.SemaphoreType.DMA,
    ],
)
def _layer_sc(qp, qwep, khbm, vhbm, bsrc, bdst, bw, cnt_hbm,
              u_out, d_out, tw_out, sc_out,
              big, mrep, vecA, vecB, src_v, dst_v, w_v, sc_v, kv, cnt_v, sem):
    wid = _wid()
    ebase = wid * ECAP
    iota = lax.iota(jnp.int32, 16)
    pltpu.sync_copy(cnt_hbm.at[pl.ds(_al(wid * 16), 16)], cnt_v)
    nchunks = jnp.max(cnt_v[...]) // C

    pltpu.sync_copy(qp.at[pl.ds(_al(wid * R * 16), R * 16)], big.at[pl.ds(0, R * 16)])
    pltpu.sync_copy(qwep.at[pl.ds(_al(wid * R), R)], vecA.at[pl.ds(0, R)])

    negv = jnp.full((16,), NEG, jnp.float32)

    def initm(i, _):
        mrep[pl.ds(i * 16, 16)] = negv
        return 0

    lax.fori_loop(0, (16 * ACC) // 16, initm, 0)

    def gather_rows(table, idx_ref, dst_ref):
        hs = []
        for j in range(C // 128):
            hs.append(pltpu.async_copy(
                table.at[idx_ref.at[pl.ds(j * 128, 128)]],
                dst_ref.at[pl.ds(j * 128, 128)], sem))
        for h in hs:
            h.wait()

    def chunkA(ci, _):
        base = ci * C
        pltpu.sync_copy(bsrc.at[pl.ds(_al(ebase + base), C)], src_v)
        pltpu.sync_copy(bdst.at[pl.ds(_al(ebase + base), C)], dst_v)
        pltpu.sync_copy(bw.at[pl.ds(_al(ebase + base), C)], w_v)
        gather_rows(khbm, src_v, kv)

        def grp(g, _):
            sl = pl.ds(g * 16, 16)
            d16 = dst_v[sl]
            eids = jnp.full((16,), g * 16, jnp.int32) + iota
            s = w_v[sl] * plsc.load_gather(vecA, [d16])
            d16x = d16 * 16
            for f in range(16):
                qf = plsc.load_gather(big, [d16x + f])
                kf = plsc.load_gather(kv, [eids, jnp.full((16,), f, jnp.int32)])
                s = s + qf * kf
            s = s * _SCALE
            addr = iota * ACC + d16
            old = plsc.load_gather(mrep, [addr])
            plsc.store_scatter(mrep, [addr], jnp.maximum(old, s))
            sc_v[sl] = s
            return 0

        lax.fori_loop(0, C // 16, grp, 0)
        pltpu.sync_copy(sc_v, sc_out.at[pl.ds(_al(ebase + base), C)])
        return 0

    lax.fori_loop(0, nchunks, chunkA, 0)

    def redm(i, _):
        sl = pl.ds(i * 16, 16)
        acc = mrep[sl]
        for r in range(1, 16):
            acc = jnp.maximum(acc, mrep[pl.ds(r * ACC + i * 16, 16)])
        mrep[sl] = acc
        return 0

    lax.fori_loop(0, ACC // 16, redm, 0)

    zf = jnp.zeros((16,), jnp.float32)

    def zbig(i, _):
        big[pl.ds(i * 16, 16)] = zf
        return 0

    lax.fori_loop(0, (ACC * 16) // 16, zbig, 0)

    def zvec(i, _):
        vecA[pl.ds(i * 16, 16)] = zf
        vecB[pl.ds(i * 16, 16)] = zf
        return 0

    lax.fori_loop(0, ACC // 16, zvec, 0)

    def chunkB(ci, _):
        base = ci * C
        pltpu.sync_copy(bsrc.at[pl.ds(_al(ebase + base), C)], src_v)
        pltpu.sync_copy(bdst.at[pl.ds(_al(ebase + base), C)], dst_v)
        pltpu.sync_copy(bw.at[pl.ds(_al(ebase + base), C)], w_v)
        pltpu.sync_copy(sc_out.at[pl.ds(_al(ebase + base), C)], sc_v)
        gather_rows(vhbm, src_v, kv)

        def grp(g, _):
            sl = pl.ds(g * 16, 16)
            d16 = dst_v[sl]
            eids = jnp.full((16,), g * 16, jnp.int32) + iota
            mm = plsc.load_gather(mrep, [d16])
            p = jnp.exp(sc_v[sl] - mm)
            plsc.addupdate_scatter(vecA, [d16], p)
            plsc.addupdate_scatter(vecB, [d16], p * w_v[sl])
            d16x = d16 * 16
            for f in range(16):
                vf = plsc.load_gather(kv, [eids, jnp.full((16,), f, jnp.int32)])
                plsc.addupdate_scatter(big, [d16x + f], p * vf)
            return 0

        lax.fori_loop(0, C // 16, grp, 0)
        return 0

    lax.fori_loop(0, nchunks, chunkB, 0)

    pltpu.sync_copy(big.at[pl.ds(0, R * 16)], u_out.at[pl.ds(_al(wid * R * 16), R * 16)])
    pltpu.sync_copy(vecA.at[pl.ds(0, R)], d_out.at[pl.ds(_al(wid * R), R)])
    pltpu.sync_copy(vecB.at[pl.ds(0, R)], tw_out.at[pl.ds(_al(wid * R), R)])


RNGE = 3136


@functools.partial(
    pl.kernel,
    mesh=_MESH,
    compiler_params=_SC_PARAMS,
    out_type=[
        jax.ShapeDtypeStruct((NW * NTX,), jnp.float32),
        jax.ShapeDtypeStruct((NW * NTX,), jnp.float32),
    ],
    scratch_types=[
        pltpu.VMEM((NTX + 16,), jnp.float32),
        pltpu.VMEM((NTX + 16,), jnp.float32),
        pltpu.VMEM((RNGE,), jnp.float32),
        pltpu.VMEM((RNGE,), jnp.int32),
    ],
)
def _seg_mean_sc(z_hbm, tx_hbm, sums_hbm, cnts_hbm, sums_v, cnts_v, z_v, tx_v):
    wid = _wid()
    base = wid * RNGE

    def zero_body(i, _):
        sl = pl.ds(i * 16, 16)
        sums_v[sl] = jnp.zeros((16,), jnp.float32)
        cnts_v[sl] = jnp.zeros((16,), jnp.float32)
        return 0

    lax.fori_loop(0, (NTX + 16) // 16, zero_body, 0)

    pltpu.sync_copy(z_hbm.at[pl.ds(_al(base), RNGE)], z_v)
    pltpu.sync_copy(tx_hbm.at[pl.ds(_al(base), RNGE)], tx_v)

    ones = jnp.ones((16,), jnp.float32)

    def body(i, _):
        sl = pl.ds(i * 16, 16)
        idx = tx_v[sl]
        plsc.addupdate_scatter(sums_v, [idx], z_v[sl])
        plsc.addupdate_scatter(cnts_v, [idx], ones)
        return 0

    lax.fori_loop(0, RNGE // 16, body, 0)

    pltpu.sync_copy(sums_v.at[pl.ds(0, NTX)], sums_hbm.at[pl.ds(_al(wid * NTX), NTX)])
    pltpu.sync_copy(cnts_v.at[pl.ds(0, NTX)], cnts_hbm.at[pl.ds(_al(wid * NTX), NTX)])


_BLK = 2048
_GRID = NP2 // _BLK

_FREQS = np.exp(-np.log(10000.0) * np.arange(NU // 2) / (NU // 2)).astype(np.float32)


def _row_spec(width):
    return pl.BlockSpec((_BLK, width), lambda i: (i, 0))


def _w_spec(shape):
    return pl.BlockSpec(shape, lambda i: tuple(0 for _ in shape))


def _prologue_body(x_ref, t_ref, W_in, b_in, Wt0, Wy0, Wq0, Wk0, Wv0, We0,
                   val_o, temb_o, q_o, k_o, v_o, qwe_o):
    x = x_ref[...]
    tf = t_ref[...]
    args = jnp.concatenate([tf * float(_FREQS[j]) for j in range(NU // 2)],
                           axis=1)
    temb = jnp.concatenate([jnp.sin(args), jnp.cos(args)], axis=-1)
    val0 = x * W_in[0][None, :] + b_in[0][None, :]
    h = val0 + jnp.dot(temb, Wt0[...]) + x * Wy0[0][None, :]
    q = jnp.dot(h, Wq0[...])
    val_o[...] = val0
    temb_o[...] = temb
    q_o[...] = q
    k_o[...] = jnp.dot(h, Wk0[...])
    v_o[...] = jnp.dot(h, Wv0[...])
    qwe_o[...] = jnp.sum(q * We0[0][None, :], axis=1, keepdims=True)


_prologue_tc = pl.pallas_call(
    _prologue_body,
    grid=(_GRID,),
    in_specs=[_row_spec(1), _row_spec(1),
              _w_spec((1, NU)), _w_spec((1, NU)), _w_spec((NU, NU)),
              _w_spec((1, NU)), _w_spec((NU, NU)), _w_spec((NU, NU)),
              _w_spec((NU, NU)), _w_spec((1, NU))],
    out_specs=[_row_spec(NU), _row_spec(NU), _row_spec(NU), _row_spec(NU),
               _row_spec(NU), _row_spec(1)],
    out_shape=[jax.ShapeDtypeStruct((NP2, NU), jnp.float32)] * 5
    + [jax.ShapeDtypeStruct((NP2, 1), jnp.float32)],
)


def _post_common(val_ref, u_ref, tw_ref, d_ref, We_c, W1, W2):
    agg = (u_ref[...] + tw_ref[...] * We_c[0][None, :]) / (d_ref[...] + 1e-16)
    h2 = val_ref[...] + agg
    ff = jnp.dot(jax.nn.relu(jnp.dot(h2, W1[...])), W2[...])
    return h2 + ff


def _mid_body(val_ref, u_ref, tw_ref, d_ref, temb_ref, x_ref,
              We_c, W1, W2, Wt_n, Wy_n, Wq_n, Wk_n, Wv_n, We_n,
              val_o, q_o, k_o, v_o, qwe_o):
    v2 = _post_common(val_ref, u_ref, tw_ref, d_ref, We_c, W1, W2)
    h = v2 + jnp.dot(temb_ref[...], Wt_n[...]) + x_ref[...] * Wy_n[0][None, :]
    q = jnp.dot(h, Wq_n[...])
    val_o[...] = v2
    q_o[...] = q
    k_o[...] = jnp.dot(h, Wk_n[...])
    v_o[...] = jnp.dot(h, Wv_n[...])
    qwe_o[...] = jnp.sum(q * We_n[0][None, :], axis=1, keepdims=True)


_mid_tc = pl.pallas_call(
    _mid_body,
    grid=(_GRID,),
    in_specs=[_row_spec(NU), _row_spec(NU), _row_spec(1), _row_spec(1),
              _row_spec(NU), _row_spec(1),
              _w_spec((1, NU)), _w_spec((NU, 4 * NU)), _w_spec((4 * NU, NU)),
              _w_spec((NU, NU)), _w_spec((1, NU)), _w_spec((NU, NU)),
              _w_spec((NU, NU)), _w_spec((NU, NU)), _w_spec((1, NU))],
    out_specs=[_row_spec(NU), _row_spec(NU), _row_spec(NU), _row_spec(NU),
               _row_spec(1)],
    out_shape=[jax.ShapeDtypeStruct((NP2, NU), jnp.float32)] * 4
    + [jax.ShapeDtypeStruct((NP2, 1), jnp.float32)],
)


def _final_body(val_ref, u_ref, tw_ref, d_ref, We_c, W1, W2, W_out, b_out, z_o):
    v2 = _post_common(val_ref, u_ref, tw_ref, d_ref, We_c, W1, W2)
    z_o[...] = jnp.dot(v2, W_out[...]) + b_out[0][None, :]


_final_tc = pl.pallas_call(
    _final_body,
    grid=(_GRID,),
    in_specs=[_row_spec(NU), _row_spec(NU), _row_spec(1), _row_spec(1),
              _w_spec((1, NU)), _w_spec((NU, 4 * NU)), _w_spec((4 * NU, NU)),
              _w_spec((NU, 1)), _w_spec((1, 1))],
    out_specs=[_row_spec(1)],
    out_shape=[jax.ShapeDtypeStruct((NP2, 1), jnp.float32)],
)


def kernel(x, t, edge_index, edge_weight, batch, transmitters_index,
           W_in, b_in, Wq, Wk, Wv, We, Wt, Wy, W1, W2, W_out, b_out):
    pad = NP2 - N
    xp = jnp.pad(x, ((0, pad), (0, 0)))
    tp = jnp.pad(t.astype(jnp.float32)[:, None], ((0, pad), (0, 0)))

    src = edge_index[0]
    dst = edge_index[1]
    bsrc, bdst, bw, cnts = _bucket_sc(src, dst, edge_weight)

    b_in2 = b_in[None, :]
    b_out2 = b_out[None, :]

    val, temb, Q, K, V, qWe = _prologue_tc(
        xp, tp, W_in, b_in2, Wt[0, 0], Wy[0, 0], Wq[0, 0], Wk[0, 0],
        Wv[0, 0], We[0, 0])

    for li in range(NB * NL):
        b, l = divmod(li, NL)
        U, dd, tw, _ = _layer_sc(
            Q.reshape(-1), qWe[:, 0], K, V, bsrc, bdst, bw, cnts)
        U = U.reshape(NP2, NU)
        dd = dd[:, None]
        tw = tw[:, None]
        if li < NB * NL - 1:
            bn, ln = divmod(li + 1, NL)
            val, Q, K, V, qWe = _mid_tc(
                val, U, tw, dd, temb, xp,
                We[b, l], W1[b, l], W2[b, l],
                Wt[bn, ln], Wy[bn, ln], Wq[bn, ln], Wk[bn, ln], Wv[bn, ln],
                We[bn, ln])
        else:
            z = _final_tc(val, U, tw, dd, We[b, l], W1[b, l], W2[b, l],
                          W_out, b_out2)[0]
    txp = jnp.pad(transmitters_index, (0, pad), constant_values=NTX)
    sums_p, cnts_p = _seg_mean_sc(z[:, 0], txp)
    sums = sums_p.reshape(NW, NTX).sum(axis=0)
    counts = cnts_p.reshape(NW, NTX).sum(axis=0)
    return (sums / jnp.maximum(counts, 1.0))[:, None]


# speedup vs baseline: 20.3801x; 1.3024x over previous
import functools

import jax
import jax.numpy as jnp
import numpy as np
from jax import lax
from jax.experimental import pallas as pl
from jax.experimental.pallas import tpu as pltpu
from jax.experimental.pallas import tpu_sc as plsc

N = 100000
E = 1600000
NF = 1
NU = 16
NB = 3
NL = 3
NSTEPS = 100
NTX = 10000

NW = 32
R = 3136
NP2 = NW * R
ACC = R + 16
DUMMY = R
ECAP = 66048
C = 384
CH0 = 2000
STG = 3072
NEG = -3.0e38

_SCALE = 1.0 / np.sqrt(float(NU))

_MESH = plsc.VectorSubcoreMesh(core_axis_name="c", subcore_axis_name="s")
_SC_PARAMS = pltpu.CompilerParams(needs_layout_passes=False, use_tc_tiling_on_sc=False)


def _wid():
    return lax.axis_index("s") * 2 + lax.axis_index("c")


def _al(x):
    return pl.multiple_of(x, 8)


@functools.partial(
    pl.kernel,
    mesh=_MESH,
    compiler_params=_SC_PARAMS,
    out_type=[
        jax.ShapeDtypeStruct((NW * 3 * ECAP,), jnp.int32),
        jax.ShapeDtypeStruct((NW * 16,), jnp.int32),
    ],
    scratch_types=[
        pltpu.VMEM((CH0,), jnp.int32),
        pltpu.VMEM((CH0,), jnp.int32),
        pltpu.VMEM((CH0,), jnp.float32),
        pltpu.VMEM((STG,), jnp.int32),
        pltpu.VMEM((STG,), jnp.int32),
        pltpu.VMEM((STG,), jnp.int32),
    ],
)
def _bucket_sc(src_hbm, dst_hbm, w_hbm, brec, cnt_hbm,
               src_v, dst_v, w_v, ssrc, sdst, sw):
    wid = _wid()
    ebase3 = wid * 3 * ECAP
    lo = wid * R
    hi = jnp.where(wid < NW - 1, lo + R, N)
    iota = lax.iota(jnp.int32, 16)
    zi = jnp.zeros((16,), jnp.int32)
    zf = jnp.zeros((16,), jnp.float32)
    dumv = jnp.full((16,), DUMMY, jnp.int32)

    def flush(nfl, off):
        def fl(i, _):
            s = pl.ds(i * C, C)
            ob = ebase3 + (off + i * C) * 3
            pltpu.sync_copy(ssrc.at[s], brec.at[pl.ds(_al(ob), C)])
            pltpu.sync_copy(sdst.at[s], brec.at[pl.ds(_al(ob + C), C)])
            pltpu.sync_copy(sw.at[s], brec.at[pl.ds(_al(ob + 2 * C), C)])
            return 0
        lax.fori_loop(0, nfl, fl, 0)

    def chunk(ch, carry):
        fill, off = carry
        s = pl.ds(ch * CH0, CH0)
        pltpu.sync_copy(src_hbm.at[s], src_v)
        pltpu.sync_copy(dst_hbm.at[s], dst_v)
        pltpu.sync_copy(w_hbm.at[s], w_v)

        def grp(g, fill):
            sl = pl.ds(g * 16, 16)
            d16 = dst_v[sl]
            own = (d16 >= lo) & (d16 < hi)
            onesi = jnp.where(own, 1, 0)
            pos = jnp.full((16,), fill, jnp.int32) + plsc.cumsum(onesi) - onesi
            plsc.store_scatter(ssrc, [pos], src_v[sl], mask=own)
            plsc.store_scatter(sdst, [pos], d16 - lo, mask=own)
            plsc.store_scatter(sw, [pos], plsc.bitcast(w_v[sl], jnp.int32),
                               mask=own)
            return fill + jnp.sum(onesi)

        fill = lax.fori_loop(0, CH0 // 16, grp, fill)
        nfl = fill // C
        flush(nfl, off)
        off = off + nfl * C
        rem = fill - nfl * C

        def comp(i, _):
            lane = i * 16 + iota
            srcpos = nfl * C + lane
            msk = lane < rem
            vs = plsc.load_gather(ssrc, [srcpos], mask=msk)
            plsc.store_scatter(ssrc, [lane], vs, mask=msk)
            vd = plsc.load_gather(sdst, [srcpos], mask=msk)
            plsc.store_scatter(sdst, [lane], vd, mask=msk)
            vw = plsc.load_gather(sw, [srcpos], mask=msk)
            plsc.store_scatter(sw, [lane], vw, mask=msk)
            return 0

        lax.fori_loop(0, C // 16, comp, 0)
        return rem, off

    fill, off = lax.fori_loop(0, E // CH0, chunk, (jnp.int32(0), jnp.int32(0)))

    pad = (C - fill % C) % C

    def padstep(i, _):
        lane = i * 16 + iota
        msk = lane < pad
        fpos = jnp.full((16,), fill, jnp.int32) + lane
        plsc.store_scatter(ssrc, [fpos], zi, mask=msk)
        plsc.store_scatter(sdst, [fpos], dumv, mask=msk)
        plsc.store_scatter(sw, [fpos], zi, mask=msk)
        return 0

    lax.fori_loop(0, C // 16, padstep, 0)
    fill = fill + pad
    nfl = fill // C
    flush(nfl, off)
    off = off + nfl * C

    ssrc[pl.ds(0, 16)] = jnp.full((16,), off, jnp.int32)
    pltpu.sync_copy(ssrc.at[pl.ds(0, 16)], cnt_hbm.at[pl.ds(_al(wid * 16), 16)])


@functools.partial(
    pl.kernel,
    mesh=_MESH,
    compiler_params=_SC_PARAMS,
    out_type=[
        jax.ShapeDtypeStruct((NP2 * 16,), jnp.float32),
        jax.ShapeDtypeStruct((NP2,), jnp.float32),
        jax.ShapeDtypeStruct((NP2,), jnp.float32),
        jax.ShapeDtypeStruct((NW * ECAP,), jnp.float32),
    ],
    scratch_types=[
        pltpu.VMEM((ACC * 16,), jnp.float32),
        pltpu.VMEM((16 * ACC,), jnp.float32),
        pltpu.VMEM((ACC,), jnp.float32),
        pltpu.VMEM((ACC,), jnp.float32),
        pltpu.VMEM((3 * 3 * C,), jnp.int32),
        pltpu.VMEM((2 * C, 16), jnp.float32),
        pltpu.VMEM((3 * C,), jnp.float32),
        pltpu.VMEM((16,), jnp.int32),
        pltpu.SemaphoreType.DMA,
        pltpu.SemaphoreType.DMA,
    ],
)
def _layer_sc(qp, qwep, khbm, vhbm, brec, cnt_hbm,
              u_out, d_out, tw_out, sc_out,
              big, mrep, vecA, vecB, rec, kv, scv, cnt_v, rsem, gsem):
    wid = _wid()
    ebase = wid * ECAP
    ebase3 = wid * 3 * ECAP
    iota = lax.iota(jnp.int32, 16)
    pltpu.sync_copy(cnt_hbm.at[pl.ds(_al(wid * 16), 16)], cnt_v)
    nchunks = jnp.max(cnt_v[...]) // C

    pltpu.sync_copy(qp.at[pl.ds(_al(wid * R * 16), R * 16)],
                    big.at[pl.ds(0, R * 16)])
    pltpu.sync_copy(qwep.at[pl.ds(_al(wid * R), R)], vecA.at[pl.ds(0, R)])

    negv = jnp.full((16,), NEG, jnp.float32)

    def initm(i, _):
        mrep[pl.ds(i * 16, 16)] = negv
        return 0

    lax.fori_loop(0, (16 * ACC) // 16, initm, 0)

    def issue_rec(j, pass_b):
        slot = lax.rem(j, 3)
        pltpu.async_copy(brec.at[pl.ds(_al(ebase3 + j * 3 * C), 3 * C)],
                         rec.at[pl.ds(slot * 3 * C, 3 * C)], rsem)
        if pass_b:
            pltpu.async_copy(sc_out.at[pl.ds(_al(ebase + j * C), C)],
                             scv.at[pl.ds(slot * C, C)], rsem)

    def wait_rec(pass_b):
        pltpu.make_async_copy(brec.at[pl.ds(0, 3 * C)],
                              rec.at[pl.ds(0, 3 * C)], rsem).wait()
        if pass_b:
            pltpu.make_async_copy(sc_out.at[pl.ds(0, C)],
                                  scv.at[pl.ds(0, C)], rsem).wait()

    def issue_gather(j, table):
        slot = lax.rem(j, 3)
        slot2 = lax.rem(j, 2)
        for sub in range(C // 128):
            pltpu.async_copy(
                table.at[rec.at[pl.ds(slot * 3 * C + sub * 128, 128)]],
                kv.at[pl.ds(slot2 * C + sub * 128, 128)], gsem)

    def wait_gather():
        for sub in range(C // 128):
            pltpu.make_async_copy(khbm.at[pl.ds(0, 128)],
                                  kv.at[pl.ds(0, 128)], gsem).wait()

    def run_pass(pass_b):
        table = vhbm if pass_b else khbm

        @pl.when(nchunks > 0)
        def _prologue():
            issue_rec(0, pass_b)
            wait_rec(pass_b)
            issue_gather(0, table)

            @pl.when(nchunks > 1)
            def _():
                issue_rec(1, pass_b)

        def body(i, _):
            @pl.when(i + 1 < nchunks)
            def _():
                wait_rec(pass_b)
                wait_gather()
                issue_gather(i + 1, table)

                @pl.when(i + 2 < nchunks)
                def __():
                    issue_rec(i + 2, pass_b)

            @pl.when(i + 1 >= nchunks)
            def _():
                wait_gather()

            slot = lax.rem(i, 3)
            slot2 = lax.rem(i, 2)
            rb = slot * 3 * C
            kb = slot2 * C
            sb = slot * C

            def grp(g, _):
                dst_sl = pl.ds(rb + C + g * 16, 16)
                w_sl = pl.ds(rb + 2 * C + g * 16, 16)
                d16 = rec[dst_sl]
                w16 = plsc.bitcast(rec[w_sl], jnp.float32)
                eids = jnp.full((16,), kb + g * 16, jnp.int32) + iota
                if not pass_b:
                    s = w16 * plsc.load_gather(vecA, [d16])
                    d16x = d16 * 16
                    for f in range(16):
                        qf = plsc.load_gather(big, [d16x + f])
                        kf = plsc.load_gather(
                            kv, [eids, jnp.full((16,), f, jnp.int32)])
                        s = s + qf * kf
                    s = s * _SCALE
                    addr = iota * ACC + d16
                    old = plsc.load_gather(mrep, [addr])
                    plsc.store_scatter(mrep, [addr], jnp.maximum(old, s))
                    scv[pl.ds(sb + g * 16, 16)] = s
                else:
                    mm = plsc.load_gather(mrep, [d16])
                    p = jnp.exp(scv[pl.ds(sb + g * 16, 16)] - mm)
                    plsc.addupdate_scatter(vecA, [d16], p)
                    plsc.addupdate_scatter(vecB, [d16], p * w16)
                    d16x = d16 * 16
                    for f in range(16):
                        vf = plsc.load_gather(
                            kv, [eids, jnp.full((16,), f, jnp.int32)])
                        plsc.addupdate_scatter(big, [d16x + f], p * vf)
                return 0

            lax.fori_loop(0, C // 16, grp, 0)
            if not pass_b:
                pltpu.sync_copy(scv.at[pl.ds(sb, C)],
                                sc_out.at[pl.ds(_al(ebase + i * C), C)])
            return 0

        lax.fori_loop(0, nchunks, body, 0)

    run_pass(False)

    def redm(i, _):
        sl = pl.ds(i * 16, 16)
        acc = mrep[sl]
        for r in range(1, 16):
            acc = jnp.maximum(acc, mrep[pl.ds(r * ACC + i * 16, 16)])
        mrep[sl] = acc
        return 0

    lax.fori_loop(0, ACC // 16, redm, 0)

    zf = jnp.zeros((16,), jnp.float32)

    def zbig(i, _):
        big[pl.ds(i * 16, 16)] = zf
        return 0

    lax.fori_loop(0, (ACC * 16) // 16, zbig, 0)

    def zvec(i, _):
        vecA[pl.ds(i * 16, 16)] = zf
        vecB[pl.ds(i * 16, 16)] = zf
        return 0

    lax.fori_loop(0, ACC // 16, zvec, 0)

    run_pass(True)

    pltpu.sync_copy(big.at[pl.ds(0, R * 16)],
                    u_out.at[pl.ds(_al(wid * R * 16), R * 16)])
    pltpu.sync_copy(vecA.at[pl.ds(0, R)], d_out.at[pl.ds(_al(wid * R), R)])
    pltpu.sync_copy(vecB.at[pl.ds(0, R)], tw_out.at[pl.ds(_al(wid * R), R)])


RNGE = 3136


@functools.partial(
    pl.kernel,
    mesh=_MESH,
    compiler_params=_SC_PARAMS,
    out_type=[
        jax.ShapeDtypeStruct((NW * NTX,), jnp.float32),
        jax.ShapeDtypeStruct((NW * NTX,), jnp.float32),
    ],
    scratch_types=[
        pltpu.VMEM((NTX + 16,), jnp.float32),
        pltpu.VMEM((NTX + 16,), jnp.float32),
        pltpu.VMEM((RNGE,), jnp.float32),
        pltpu.VMEM((RNGE,), jnp.int32),
    ],
)
def _seg_mean_sc(z_hbm, tx_hbm, sums_hbm, cnts_hbm, sums_v, cnts_v, z_v, tx_v):
    wid = _wid()
    base = wid * RNGE

    def zero_body(i, _):
        sl = pl.ds(i * 16, 16)
        sums_v[sl] = jnp.zeros((16,), jnp.float32)
        cnts_v[sl] = jnp.zeros((16,), jnp.float32)
        return 0

    lax.fori_loop(0, (NTX + 16) // 16, zero_body, 0)

    pltpu.sync_copy(z_hbm.at[pl.ds(_al(base), RNGE)], z_v)
    pltpu.sync_copy(tx_hbm.at[pl.ds(_al(base), RNGE)], tx_v)

    ones = jnp.ones((16,), jnp.float32)

    def body(i, _):
        sl = pl.ds(i * 16, 16)
        idx = tx_v[sl]
        plsc.addupdate_scatter(sums_v, [idx], z_v[sl])
        plsc.addupdate_scatter(cnts_v, [idx], ones)
        return 0

    lax.fori_loop(0, RNGE // 16, body, 0)

    pltpu.sync_copy(sums_v.at[pl.ds(0, NTX)], sums_hbm.at[pl.ds(_al(wid * NTX), NTX)])
    pltpu.sync_copy(cnts_v.at[pl.ds(0, NTX)], cnts_hbm.at[pl.ds(_al(wid * NTX), NTX)])


_BLK = 2048
_GRID = NP2 // _BLK

_FREQS = np.exp(-np.log(10000.0) * np.arange(NU // 2) / (NU // 2)).astype(np.float32)


def _row_spec(width):
    return pl.BlockSpec((_BLK, width), lambda i: (i, 0))


def _w_spec(shape):
    return pl.BlockSpec(shape, lambda i: tuple(0 for _ in shape))


def _prologue_body(x_ref, t_ref, W_in, b_in, Wt0, Wy0, Wq0, Wk0, Wv0, We0,
                   val_o, temb_o, q_o, k_o, v_o, qwe_o):
    x = x_ref[...]
    tf = t_ref[...]
    args = jnp.concatenate([tf * float(_FREQS[j]) for j in range(NU // 2)],
                           axis=1)
    temb = jnp.concatenate([jnp.sin(args), jnp.cos(args)], axis=-1)
    val0 = x * W_in[0][None, :] + b_in[0][None, :]
    h = val0 + jnp.dot(temb, Wt0[...]) + x * Wy0[0][None, :]
    q = jnp.dot(h, Wq0[...])
    val_o[...] = val0
    temb_o[...] = temb
    q_o[...] = q
    k_o[...] = jnp.dot(h, Wk0[...])
    v_o[...] = jnp.dot(h, Wv0[...])
    qwe_o[...] = jnp.sum(q * We0[0][None, :], axis=1, keepdims=True)


_prologue_tc = pl.pallas_call(
    _prologue_body,
    grid=(_GRID,),
    in_specs=[_row_spec(1), _row_spec(1),
              _w_spec((1, NU)), _w_spec((1, NU)), _w_spec((NU, NU)),
              _w_spec((1, NU)), _w_spec((NU, NU)), _w_spec((NU, NU)),
              _w_spec((NU, NU)), _w_spec((1, NU))],
    out_specs=[_row_spec(NU), _row_spec(NU), _row_spec(NU), _row_spec(NU),
               _row_spec(NU), _row_spec(1)],
    out_shape=[jax.ShapeDtypeStruct((NP2, NU), jnp.float32)] * 5
    + [jax.ShapeDtypeStruct((NP2, 1), jnp.float32)],
)


def _post_common(val_ref, u_ref, tw_ref, d_ref, We_c, W1, W2):
    agg = (u_ref[...] + tw_ref[...] * We_c[0][None, :]) / (d_ref[...] + 1e-16)
    h2 = val_ref[...] + agg
    ff = jnp.dot(jax.nn.relu(jnp.dot(h2, W1[...])), W2[...])
    return h2 + ff


def _mid_body(val_ref, u_ref, tw_ref, d_ref, temb_ref, x_ref,
              We_c, W1, W2, Wt_n, Wy_n, Wq_n, Wk_n, Wv_n, We_n,
              val_o, q_o, k_o, v_o, qwe_o):
    v2 = _post_common(val_ref, u_ref, tw_ref, d_ref, We_c, W1, W2)
    h = v2 + jnp.dot(temb_ref[...], Wt_n[...]) + x_ref[...] * Wy_n[0][None, :]
    q = jnp.dot(h, Wq_n[...])
    val_o[...] = v2
    q_o[...] = q
    k_o[...] = jnp.dot(h, Wk_n[...])
    v_o[...] = jnp.dot(h, Wv_n[...])
    qwe_o[...] = jnp.sum(q * We_n[0][None, :], axis=1, keepdims=True)


_mid_tc = pl.pallas_call(
    _mid_body,
    grid=(_GRID,),
    in_specs=[_row_spec(NU), _row_spec(NU), _row_spec(1), _row_spec(1),
              _row_spec(NU), _row_spec(1),
              _w_spec((1, NU)), _w_spec((NU, 4 * NU)), _w_spec((4 * NU, NU)),
              _w_spec((NU, NU)), _w_spec((1, NU)), _w_spec((NU, NU)),
              _w_spec((NU, NU)), _w_spec((NU, NU)), _w_spec((1, NU))],
    out_specs=[_row_spec(NU), _row_spec(NU), _row_spec(NU), _row_spec(NU),
               _row_spec(1)],
    out_shape=[jax.ShapeDtypeStruct((NP2, NU), jnp.float32)] * 4
    + [jax.ShapeDtypeStruct((NP2, 1), jnp.float32)],
)


def _final_body(val_ref, u_ref, tw_ref, d_ref, We_c, W1, W2, W_out, b_out, z_o):
    v2 = _post_common(val_ref, u_ref, tw_ref, d_ref, We_c, W1, W2)
    z_o[...] = jnp.dot(v2, W_out[...]) + b_out[0][None, :]


_final_tc = pl.pallas_call(
    _final_body,
    grid=(_GRID,),
    in_specs=[_row_spec(NU), _row_spec(NU), _row_spec(1), _row_spec(1),
              _w_spec((1, NU)), _w_spec((NU, 4 * NU)), _w_spec((4 * NU, NU)),
              _w_spec((NU, 1)), _w_spec((1, 1))],
    out_specs=[_row_spec(1)],
    out_shape=[jax.ShapeDtypeStruct((NP2, 1), jnp.float32)],
)


def kernel(x, t, edge_index, edge_weight, batch, transmitters_index,
           W_in, b_in, Wq, Wk, Wv, We, Wt, Wy, W1, W2, W_out, b_out):
    pad = NP2 - N
    xp = jnp.pad(x, ((0, pad), (0, 0)))
    tp = jnp.pad(t.astype(jnp.float32)[:, None], ((0, pad), (0, 0)))

    src = edge_index[0]
    dst = edge_index[1]
    brec, cnts = _bucket_sc(src, dst, edge_weight)

    b_in2 = b_in[None, :]
    b_out2 = b_out[None, :]

    val, temb, Q, K, V, qWe = _prologue_tc(
        xp, tp, W_in, b_in2, Wt[0, 0], Wy[0, 0], Wq[0, 0], Wk[0, 0],
        Wv[0, 0], We[0, 0])

    for li in range(NB * NL):
        b, l = divmod(li, NL)
        U, dd, tw, _ = _layer_sc(
            Q.reshape(-1), qWe[:, 0], K, V, brec, cnts)
        U = U.reshape(NP2, NU)
        dd = dd[:, None]
        tw = tw[:, None]
        if li < NB * NL - 1:
            bn, ln = divmod(li + 1, NL)
            val, Q, K, V, qWe = _mid_tc(
                val, U, tw, dd, temb, xp,
                We[b, l], W1[b, l], W2[b, l],
                Wt[bn, ln], Wy[bn, ln], Wq[bn, ln], Wk[bn, ln], Wv[bn, ln],
                We[bn, ln])
        else:
            z = _final_tc(val, U, tw, dd, We[b, l], W1[b, l], W2[b, l],
                          W_out, b_out2)[0]
    txp = jnp.pad(transmitters_index, (0, pad), constant_values=NTX)
    sums_p, cnts_p = _seg_mean_sc(z[:, 0], txp)
    sums = sums_p.reshape(NW, NTX).sum(axis=0)
    counts = cnts_p.reshape(NW, NTX).sum(axis=0)
    return (sums / jnp.maximum(counts, 1.0))[:, None]


# speedup vs baseline: 21.2478x; 1.0426x over previous
import functools

import jax
import jax.numpy as jnp
import numpy as np
from jax import lax
from jax.experimental import pallas as pl
from jax.experimental.pallas import tpu as pltpu
from jax.experimental.pallas import tpu_sc as plsc

N = 100000
E = 1600000
NF = 1
NU = 16
NB = 3
NL = 3
NSTEPS = 100
NTX = 10000

NW = 32
R = 3136
NP2 = NW * R
ACC = R + 16
DUMMY = R
ECAP = 66048
C = 384
CH0 = 3200
STG = 3072
NEG = -3.0e38

_SCALE = 1.0 / np.sqrt(float(NU))

_MESH = plsc.VectorSubcoreMesh(core_axis_name="c", subcore_axis_name="s")
_SC_PARAMS = pltpu.CompilerParams(needs_layout_passes=False, use_tc_tiling_on_sc=False)


def _wid():
    return lax.axis_index("s") * 2 + lax.axis_index("c")


def _al(x):
    return pl.multiple_of(x, 8)


NARENA = 4
ACAP = ECAP // NARENA


@functools.partial(
    pl.kernel,
    mesh=_MESH,
    compiler_params=_SC_PARAMS,
    out_type=[
        jax.ShapeDtypeStruct((NW * 3 * ECAP,), jnp.int32),
        jax.ShapeDtypeStruct((NW * 16,), jnp.int32),
    ],
    scratch_types=[
        pltpu.VMEM((2 * CH0,), jnp.int32),
        pltpu.VMEM((2 * CH0,), jnp.int32),
        pltpu.VMEM((2 * CH0,), jnp.float32),
        pltpu.VMEM((NARENA * STG,), jnp.int32),
        pltpu.VMEM((NARENA * STG,), jnp.int32),
        pltpu.VMEM((NARENA * STG,), jnp.int32),
        pltpu.SemaphoreType.DMA,
        pltpu.SemaphoreType.DMA,
    ],
)
def _bucket_sc(src_hbm, dst_hbm, w_hbm, brec, cnt_hbm,
               src_v, dst_v, w_v, ssrc, sdst, sw, semA, semB):
    wid = _wid()
    ebase3 = wid * 3 * ECAP
    lo = wid * R
    hi = jnp.where(wid < NW - 1, lo + R, N)
    iota = lax.iota(jnp.int32, 16)
    zi = jnp.zeros((16,), jnp.int32)
    dumv = jnp.full((16,), DUMMY, jnp.int32)

    def issue(ch, slot, sem):
        s = pl.ds(_al(ch * CH0), CH0)
        d = pl.ds(slot * CH0, CH0)
        pltpu.async_copy(src_hbm.at[s], src_v.at[d], sem)
        pltpu.async_copy(dst_hbm.at[s], dst_v.at[d], sem)
        pltpu.async_copy(w_hbm.at[s], w_v.at[d], sem)

    def wait_in(sem):
        for ref in (src_v, dst_v, w_v):
            pltpu.make_async_copy(src_hbm.at[pl.ds(0, CH0)],
                                  ref.at[pl.ds(0, CH0)], sem).wait()

    def flush_arena(a, nfl, off):
        def fl(i, _):
            sb = a * STG
            s = pl.ds(sb + i * C, C)
            ob = ebase3 + a * 3 * ACAP + (off + i * C) * 3
            pltpu.sync_copy(ssrc.at[s], brec.at[pl.ds(_al(ob), C)])
            pltpu.sync_copy(sdst.at[s], brec.at[pl.ds(_al(ob + C), C)])
            pltpu.sync_copy(sw.at[s], brec.at[pl.ds(_al(ob + 2 * C), C)])
            return 0
        lax.fori_loop(0, nfl, fl, 0)

    def compact_arena(a, nfl, rem):
        def comp(i, _):
            sb = a * STG
            lane = i * 16 + iota
            srcpos = sb + nfl * C + lane
            dstpos = sb + lane
            msk = lane < rem
            vs = plsc.load_gather(ssrc, [srcpos], mask=msk)
            plsc.store_scatter(ssrc, [dstpos], vs, mask=msk)
            vd = plsc.load_gather(sdst, [srcpos], mask=msk)
            plsc.store_scatter(sdst, [dstpos], vd, mask=msk)
            vw = plsc.load_gather(sw, [srcpos], mask=msk)
            plsc.store_scatter(sw, [dstpos], vw, mask=msk)
            return 0
        lax.fori_loop(0, C // 16, comp, 0)

    def process(slot, carry):
        fills, offs = carry
        nf, no = [], []
        for a in range(NARENA):
            fill = fills[a]

            def grp(g, fill, a=a):
                sl = pl.ds(slot * CH0 + (g * NARENA + a) * 16, 16)
                d16 = dst_v[sl]
                own = (d16 >= lo) & (d16 < hi)
                onesi = jnp.where(own, 1, 0)
                pos = (jnp.full((16,), a * STG + fill, jnp.int32)
                       + plsc.cumsum(onesi) - onesi)
                plsc.store_scatter(ssrc, [pos], src_v[sl], mask=own)
                plsc.store_scatter(sdst, [pos], d16 - lo, mask=own)
                plsc.store_scatter(sw, [pos],
                                   plsc.bitcast(w_v[sl], jnp.int32), mask=own)
                return fill + jnp.sum(onesi)

            fill = lax.fori_loop(0, CH0 // 16 // NARENA, grp, fill)
            nfl = fill // C
            flush_arena(a, nfl, offs[a])
            rem = fill - nfl * C
            compact_arena(a, nfl, rem)
            nf.append(rem)
            no.append(offs[a] + nfl * C)
        return tuple(nf), tuple(no)

    z4 = tuple(jnp.int32(0) for _ in range(NARENA))

    def two_chunks(k, carry):
        issue(2 * k + 1, 1, semB)
        wait_in(semA)
        carry = process(0, carry)

        @pl.when(k < E // CH0 // 2 - 1)
        def _():
            issue(2 * k + 2, 0, semA)

        wait_in(semB)
        carry = process(1, carry)
        return carry

    issue(0, 0, semA)
    fills, offs = lax.fori_loop(0, E // CH0 // 2, two_chunks, (z4, z4))

    cvec = zi
    for a in range(NARENA):
        fill, off = fills[a], offs[a]
        pad = (C - fill % C) % C

        def padstep(i, _, a=a, fill=fill, pad=pad):
            lane = i * 16 + iota
            msk = lane < pad
            fpos = jnp.full((16,), a * STG + fill, jnp.int32) + lane
            plsc.store_scatter(ssrc, [fpos], zi, mask=msk)
            plsc.store_scatter(sdst, [fpos], dumv, mask=msk)
            plsc.store_scatter(sw, [fpos], zi, mask=msk)
            return 0

        lax.fori_loop(0, C // 16, padstep, 0)
        fill = fill + pad
        nfl = fill // C
        flush_arena(a, nfl, off)
        cvec = jnp.where(iota == a, jnp.full((16,), off + nfl * C, jnp.int32),
                         cvec)

    ssrc[pl.ds(0, 16)] = cvec
    pltpu.sync_copy(ssrc.at[pl.ds(0, 16)], cnt_hbm.at[pl.ds(_al(wid * 16), 16)])


@functools.partial(
    pl.kernel,
    mesh=_MESH,
    compiler_params=_SC_PARAMS,
    out_type=[
        jax.ShapeDtypeStruct((NP2 * 16,), jnp.float32),
        jax.ShapeDtypeStruct((NP2,), jnp.float32),
        jax.ShapeDtypeStruct((NP2,), jnp.float32),
        jax.ShapeDtypeStruct((NW * ECAP,), jnp.float32),
    ],
    scratch_types=[
        pltpu.VMEM((ACC * 16,), jnp.float32),
        pltpu.VMEM((16 * ACC,), jnp.float32),
        pltpu.VMEM((ACC,), jnp.float32),
        pltpu.VMEM((ACC,), jnp.float32),
        pltpu.VMEM((3 * 3 * C,), jnp.int32),
        pltpu.VMEM((2 * C, 16), jnp.float32),
        pltpu.VMEM((3 * C,), jnp.float32),
        pltpu.VMEM((16,), jnp.int32),
        pltpu.SemaphoreType.DMA,
        pltpu.SemaphoreType.DMA,
    ],
)
def _layer_sc(qp, qwep, khbm, vhbm, brec, cnt_hbm,
              u_out, d_out, tw_out, sc_out,
              big, mrep, vecA, vecB, rec, kv, scv, cnt_v, rsem, gsem):
    wid = _wid()
    ebase = wid * ECAP
    ebase3 = wid * 3 * ECAP
    iota = lax.iota(jnp.int32, 16)
    pltpu.sync_copy(cnt_hbm.at[pl.ds(_al(wid * 16), 16)], cnt_v)
    cntrow = cnt_v[...]

    pltpu.sync_copy(qp.at[pl.ds(_al(wid * R * 16), R * 16)],
                    big.at[pl.ds(0, R * 16)])
    pltpu.sync_copy(qwep.at[pl.ds(_al(wid * R), R)], vecA.at[pl.ds(0, R)])

    negv = jnp.full((16,), NEG, jnp.float32)

    def initm(i, _):
        mrep[pl.ds(i * 16, 16)] = negv
        return 0

    lax.fori_loop(0, (16 * ACC) // 16, initm, 0)

    def issue_rec(j, pass_b, rbase, sbase):
        slot = lax.rem(j, 3)
        pltpu.async_copy(brec.at[pl.ds(_al(rbase + j * 3 * C), 3 * C)],
                         rec.at[pl.ds(slot * 3 * C, 3 * C)], rsem)
        if pass_b:
            pltpu.async_copy(sc_out.at[pl.ds(_al(sbase + j * C), C)],
                             scv.at[pl.ds(slot * C, C)], rsem)

    def wait_rec(pass_b):
        pltpu.make_async_copy(brec.at[pl.ds(0, 3 * C)],
                              rec.at[pl.ds(0, 3 * C)], rsem).wait()
        if pass_b:
            pltpu.make_async_copy(sc_out.at[pl.ds(0, C)],
                                  scv.at[pl.ds(0, C)], rsem).wait()

    def issue_gather(j, table):
        slot = lax.rem(j, 3)
        slot2 = lax.rem(j, 2)
        for sub in range(C // 128):
            pltpu.async_copy(
                table.at[rec.at[pl.ds(slot * 3 * C + sub * 128, 128)]],
                kv.at[pl.ds(slot2 * C + sub * 128, 128)], gsem)

    def wait_gather():
        for sub in range(C // 128):
            pltpu.make_async_copy(khbm.at[pl.ds(0, 128)],
                                  kv.at[pl.ds(0, 128)], gsem).wait()

    def run_pass(pass_b, arena):
        table = vhbm if pass_b else khbm
        rbase = ebase3 + arena * 3 * ACAP
        sbase = ebase + arena * ACAP
        nchunks = jnp.max(jnp.where(iota == arena, cntrow, 0)) // C

        @pl.when(nchunks > 0)
        def _prologue():
            issue_rec(0, pass_b, rbase, sbase)
            wait_rec(pass_b)
            issue_gather(0, table)

            @pl.when(nchunks > 1)
            def _():
                issue_rec(1, pass_b, rbase, sbase)

        def body(i, _):
            @pl.when(i + 1 < nchunks)
            def _():
                wait_rec(pass_b)
                wait_gather()
                issue_gather(i + 1, table)

                @pl.when(i + 2 < nchunks)
                def __():
                    issue_rec(i + 2, pass_b, rbase, sbase)

            @pl.when(i + 1 >= nchunks)
            def _():
                wait_gather()

            slot = lax.rem(i, 3)
            slot2 = lax.rem(i, 2)
            rb = slot * 3 * C
            kb = slot2 * C
            sb = slot * C

            def grp(g, _):
                dst_sl = pl.ds(rb + C + g * 16, 16)
                w_sl = pl.ds(rb + 2 * C + g * 16, 16)
                d16 = rec[dst_sl]
                w16 = plsc.bitcast(rec[w_sl], jnp.float32)
                eids = jnp.full((16,), kb + g * 16, jnp.int32) + iota
                if not pass_b:
                    s = w16 * plsc.load_gather(vecA, [d16])
                    d16x = d16 * 16
                    for f in range(16):
                        qf = plsc.load_gather(big, [d16x + f])
                        kf = plsc.load_gather(
                            kv, [eids, jnp.full((16,), f, jnp.int32)])
                        s = s + qf * kf
                    s = s * _SCALE
                    addr = iota * ACC + d16
                    old = plsc.load_gather(mrep, [addr])
                    plsc.store_scatter(mrep, [addr], jnp.maximum(old, s))
                    scv[pl.ds(sb + g * 16, 16)] = s
                else:
                    mm = plsc.load_gather(mrep, [d16])
                    p = jnp.exp(scv[pl.ds(sb + g * 16, 16)] - mm)
                    plsc.addupdate_scatter(vecA, [d16], p)
                    plsc.addupdate_scatter(vecB, [d16], p * w16)
                    d16x = d16 * 16
                    for f in range(16):
                        vf = plsc.load_gather(
                            kv, [eids, jnp.full((16,), f, jnp.int32)])
                        plsc.addupdate_scatter(big, [d16x + f], p * vf)
                return 0

            lax.fori_loop(0, C // 16, grp, 0)
            if not pass_b:
                pltpu.sync_copy(scv.at[pl.ds(sb, C)],
                                sc_out.at[pl.ds(_al(sbase + i * C), C)])
            return 0

        lax.fori_loop(0, nchunks, body, 0)

    for _a in range(NARENA):
        run_pass(False, _a)

    def redm(i, _):
        sl = pl.ds(i * 16, 16)
        acc = mrep[sl]
        for r in range(1, 16):
            acc = jnp.maximum(acc, mrep[pl.ds(r * ACC + i * 16, 16)])
        mrep[sl] = acc
        return 0

    lax.fori_loop(0, ACC // 16, redm, 0)

    zf = jnp.zeros((16,), jnp.float32)

    def zbig(i, _):
        big[pl.ds(i * 16, 16)] = zf
        return 0

    lax.fori_loop(0, (ACC * 16) // 16, zbig, 0)

    def zvec(i, _):
        vecA[pl.ds(i * 16, 16)] = zf
        vecB[pl.ds(i * 16, 16)] = zf
        return 0

    lax.fori_loop(0, ACC // 16, zvec, 0)

    for _a in range(NARENA):
        run_pass(True, _a)

    pltpu.sync_copy(big.at[pl.ds(0, R * 16)],
                    u_out.at[pl.ds(_al(wid * R * 16), R * 16)])
    pltpu.sync_copy(vecA.at[pl.ds(0, R)], d_out.at[pl.ds(_al(wid * R), R)])
    pltpu.sync_copy(vecB.at[pl.ds(0, R)], tw_out.at[pl.ds(_al(wid * R), R)])


RNGE = 3136


@functools.partial(
    pl.kernel,
    mesh=_MESH,
    compiler_params=_SC_PARAMS,
    out_type=[
        jax.ShapeDtypeStruct((NW * NTX,), jnp.float32),
        jax.ShapeDtypeStruct((NW * NTX,), jnp.float32),
    ],
    scratch_types=[
        pltpu.VMEM((NTX + 16,), jnp.float32),
        pltpu.VMEM((NTX + 16,), jnp.float32),
        pltpu.VMEM((RNGE,), jnp.float32),
        pltpu.VMEM((RNGE,), jnp.int32),
    ],
)
def _seg_mean_sc(z_hbm, tx_hbm, sums_hbm, cnts_hbm, sums_v, cnts_v, z_v, tx_v):
    wid = _wid()
    base = wid * RNGE

    def zero_body(i, _):
        sl = pl.ds(i * 16, 16)
        sums_v[sl] = jnp.zeros((16,), jnp.float32)
        cnts_v[sl] = jnp.zeros((16,), jnp.float32)
        return 0

    lax.fori_loop(0, (NTX + 16) // 16, zero_body, 0)

    pltpu.sync_copy(z_hbm.at[pl.ds(_al(base), RNGE)], z_v)
    pltpu.sync_copy(tx_hbm.at[pl.ds(_al(base), RNGE)], tx_v)

    ones = jnp.ones((16,), jnp.float32)

    def body(i, _):
        sl = pl.ds(i * 16, 16)
        idx = tx_v[sl]
        plsc.addupdate_scatter(sums_v, [idx], z_v[sl])
        plsc.addupdate_scatter(cnts_v, [idx], ones)
        return 0

    lax.fori_loop(0, RNGE // 16, body, 0)

    pltpu.sync_copy(sums_v.at[pl.ds(0, NTX)], sums_hbm.at[pl.ds(_al(wid * NTX), NTX)])
    pltpu.sync_copy(cnts_v.at[pl.ds(0, NTX)], cnts_hbm.at[pl.ds(_al(wid * NTX), NTX)])


_BLK = 2048
_GRID = NP2 // _BLK

_FREQS = np.exp(-np.log(10000.0) * np.arange(NU // 2) / (NU // 2)).astype(np.float32)


def _row_spec(width):
    return pl.BlockSpec((_BLK, width), lambda i: (i, 0))


def _w_spec(shape):
    return pl.BlockSpec(shape, lambda i: tuple(0 for _ in shape))


def _prologue_body(x_ref, t_ref, W_in, b_in, Wt0, Wy0, Wq0, Wk0, Wv0, We0,
                   val_o, temb_o, q_o, k_o, v_o, qwe_o):
    x = x_ref[...]
    tf = t_ref[...]
    args = jnp.concatenate([tf * float(_FREQS[j]) for j in range(NU // 2)],
                           axis=1)
    temb = jnp.concatenate([jnp.sin(args), jnp.cos(args)], axis=-1)
    val0 = x * W_in[0][None, :] + b_in[0][None, :]
    h = val0 + jnp.dot(temb, Wt0[...]) + x * Wy0[0][None, :]
    q = jnp.dot(h, Wq0[...])
    val_o[...] = val0
    temb_o[...] = temb
    q_o[...] = q
    k_o[...] = jnp.dot(h, Wk0[...])
    v_o[...] = jnp.dot(h, Wv0[...])
    qwe_o[...] = jnp.sum(q * We0[0][None, :], axis=1, keepdims=True)


_prologue_tc = pl.pallas_call(
    _prologue_body,
    grid=(_GRID,),
    in_specs=[_row_spec(1), _row_spec(1),
              _w_spec((1, NU)), _w_spec((1, NU)), _w_spec((NU, NU)),
              _w_spec((1, NU)), _w_spec((NU, NU)), _w_spec((NU, NU)),
              _w_spec((NU, NU)), _w_spec((1, NU))],
    out_specs=[_row_spec(NU), _row_spec(NU), _row_spec(NU), _row_spec(NU),
               _row_spec(NU), _row_spec(1)],
    out_shape=[jax.ShapeDtypeStruct((NP2, NU), jnp.float32)] * 5
    + [jax.ShapeDtypeStruct((NP2, 1), jnp.float32)],
)


def _post_common(val_ref, u_ref, tw_ref, d_ref, We_c, W1, W2):
    agg = (u_ref[...] + tw_ref[...] * We_c[0][None, :]) / (d_ref[...] + 1e-16)
    h2 = val_ref[...] + agg
    ff = jnp.dot(jax.nn.relu(jnp.dot(h2, W1[...])), W2[...])
    return h2 + ff


def _mid_body(val_ref, u_ref, tw_ref, d_ref, temb_ref, x_ref,
              We_c, W1, W2, Wt_n, Wy_n, Wq_n, Wk_n, Wv_n, We_n,
              val_o, q_o, k_o, v_o, qwe_o):
    v2 = _post_common(val_ref, u_ref, tw_ref, d_ref, We_c, W1, W2)
    h = v2 + jnp.dot(temb_ref[...], Wt_n[...]) + x_ref[...] * Wy_n[0][None, :]
    q = jnp.dot(h, Wq_n[...])
    val_o[...] = v2
    q_o[...] = q
    k_o[...] = jnp.dot(h, Wk_n[...])
    v_o[...] = jnp.dot(h, Wv_n[...])
    qwe_o[...] = jnp.sum(q * We_n[0][None, :], axis=1, keepdims=True)


_mid_tc = pl.pallas_call(
    _mid_body,
    grid=(_GRID,),
    in_specs=[_row_spec(NU), _row_spec(NU), _row_spec(1), _row_spec(1),
              _row_spec(NU), _row_spec(1),
              _w_spec((1, NU)), _w_spec((NU, 4 * NU)), _w_spec((4 * NU, NU)),
              _w_spec((NU, NU)), _w_spec((1, NU)), _w_spec((NU, NU)),
              _w_spec((NU, NU)), _w_spec((NU, NU)), _w_spec((1, NU))],
    out_specs=[_row_spec(NU), _row_spec(NU), _row_spec(NU), _row_spec(NU),
               _row_spec(1)],
    out_shape=[jax.ShapeDtypeStruct((NP2, NU), jnp.float32)] * 4
    + [jax.ShapeDtypeStruct((NP2, 1), jnp.float32)],
)


def _final_body(val_ref, u_ref, tw_ref, d_ref, We_c, W1, W2, W_out, b_out, z_o):
    v2 = _post_common(val_ref, u_ref, tw_ref, d_ref, We_c, W1, W2)
    z_o[...] = jnp.dot(v2, W_out[...]) + b_out[0][None, :]


_final_tc = pl.pallas_call(
    _final_body,
    grid=(_GRID,),
    in_specs=[_row_spec(NU), _row_spec(NU), _row_spec(1), _row_spec(1),
              _w_spec((1, NU)), _w_spec((NU, 4 * NU)), _w_spec((4 * NU, NU)),
              _w_spec((NU, 1)), _w_spec((1, 1))],
    out_specs=[_row_spec(1)],
    out_shape=[jax.ShapeDtypeStruct((NP2, 1), jnp.float32)],
)


def kernel(x, t, edge_index, edge_weight, batch, transmitters_index,
           W_in, b_in, Wq, Wk, Wv, We, Wt, Wy, W1, W2, W_out, b_out):
    pad = NP2 - N
    xp = jnp.pad(x, ((0, pad), (0, 0)))
    tp = jnp.pad(t.astype(jnp.float32)[:, None], ((0, pad), (0, 0)))

    src = edge_index[0]
    dst = edge_index[1]
    brec, cnts = _bucket_sc(src, dst, edge_weight)

    b_in2 = b_in[None, :]
    b_out2 = b_out[None, :]

    val, temb, Q, K, V, qWe = _prologue_tc(
        xp, tp, W_in, b_in2, Wt[0, 0], Wy[0, 0], Wq[0, 0], Wk[0, 0],
        Wv[0, 0], We[0, 0])

    for li in range(NB * NL):
        b, l = divmod(li, NL)
        U, dd, tw, _ = _layer_sc(
            Q.reshape(-1), qWe[:, 0], K, V, brec, cnts)
        U = U.reshape(NP2, NU)
        dd = dd[:, None]
        tw = tw[:, None]
        if li < NB * NL - 1:
            bn, ln = divmod(li + 1, NL)
            val, Q, K, V, qWe = _mid_tc(
                val, U, tw, dd, temb, xp,
                We[b, l], W1[b, l], W2[b, l],
                Wt[bn, ln], Wy[bn, ln], Wq[bn, ln], Wk[bn, ln], Wv[bn, ln],
                We[bn, ln])
        else:
            z = _final_tc(val, U, tw, dd, We[b, l], W1[b, l], W2[b, l],
                          W_out, b_out2)[0]
    txp = jnp.pad(transmitters_index, (0, pad), constant_values=NTX)
    sums_p, cnts_p = _seg_mean_sc(z[:, 0], txp)
    sums = sums_p.reshape(NW, NTX).sum(axis=0)
    counts = cnts_p.reshape(NW, NTX).sum(axis=0)
    return (sums / jnp.maximum(counts, 1.0))[:, None]
